# Initial kernel scaffold; baseline (speedup 1.0000x reference)
import jax
import jax.numpy as jnp
from jax.experimental import pallas as pl


def kernel(h_n_l, h_g_l, node_matrix, node_batch, W_2, W_3, fc_n_w, fc_n_b, fc_g_w, fc_g_b):
    raise NotImplementedError("write your pallas kernel here")


# speedup vs baseline: 1.2489x; 1.2489x over previous
import functools

import jax
import jax.numpy as jnp
from jax.experimental import pallas as pl


def _depth_kernel(a_ref, h_ref, hg_ref, nb_ref, w2_ref, w3_ref,
                  fcnw_ref, fcnb_ref, fcgw_ref, fcgb_ref,
                  out_ref, hg_out_ref, *, nblocks, d):
    i = pl.program_id(0)

    acc = jnp.dot(a_ref[...], h_ref[...], preferred_element_type=jnp.float32)
    z2 = jnp.dot(acc, w3_ref[...], preferred_element_type=jnp.float32)
    z1 = jnp.dot(hg_ref[...], w2_ref[...], preferred_element_type=jnp.float32)
    t_dims = (((1,), (1,)), ((), ()))
    pre = (jax.lax.dot_general(z2, fcnw_ref[:, d:], t_dims,
                               preferred_element_type=jnp.float32)
           + jax.lax.dot_general(z1, fcnw_ref[:, :d], t_dims,
                                 preferred_element_type=jnp.float32)
           + fcnb_ref[...])
    t = jnp.maximum(pre, 0.0)
    nrm = jnp.sqrt(jnp.sum(t * t, axis=-1, keepdims=True))
    t = t / jnp.maximum(nrm, 1e-12)
    out_ref[...] = t

    part = jnp.dot(nb_ref[0], t, preferred_element_type=jnp.float32)

    @pl.when(i == 0)
    def _init():
        hg_out_ref[...] = part

    @pl.when(i > 0)
    def _accum():
        hg_out_ref[...] += part

    @pl.when(i == nblocks - 1)
    def _finish():
        h_ng = hg_out_ref[...]
        z4 = jnp.dot(h_ng, w3_ref[...], preferred_element_type=jnp.float32)
        pre_g = (jax.lax.dot_general(z1, fcgw_ref[:, :d], t_dims,
                                     preferred_element_type=jnp.float32)
                 + jax.lax.dot_general(z4, fcgw_ref[:, d:], t_dims,
                                       preferred_element_type=jnp.float32)
                 + fcgb_ref[...])
        tg = jnp.maximum(pre_g, 0.0)
        nrm_g = jnp.sqrt(jnp.sum(tg * tg, axis=-1, keepdims=True))
        hg_out_ref[...] = tg / jnp.maximum(nrm_g, 1e-12)


def _one_depth(h_n, h_g, a, nb, w2, w3, fcnw, fcnb2, fcgw, fcgb2,
               *, bm, interpret=False):
    n, d = h_n.shape
    nblocks = n // bm
    grid = (nblocks,)
    const = lambda *_: tuple(0 for _ in _)
    kfn = functools.partial(_depth_kernel, nblocks=nblocks, d=d)
    return pl.pallas_call(
        kfn,
        grid=grid,
        in_specs=[
            pl.BlockSpec((bm, n), lambda i: (i, 0)),
            pl.BlockSpec((n, d), lambda i: (0, 0)),
            pl.BlockSpec((1, d), lambda i: (0, 0)),
            pl.BlockSpec((1, 1, bm), lambda i: (i, 0, 0)),
            pl.BlockSpec((d, d), lambda i: (0, 0)),
            pl.BlockSpec((d, d), lambda i: (0, 0)),
            pl.BlockSpec((d, 2 * d), lambda i: (0, 0)),
            pl.BlockSpec((1, d), lambda i: (0, 0)),
            pl.BlockSpec((d, 2 * d), lambda i: (0, 0)),
            pl.BlockSpec((1, d), lambda i: (0, 0)),
        ],
        out_specs=[
            pl.BlockSpec((bm, d), lambda i: (i, 0)),
            pl.BlockSpec((1, d), lambda i: (0, 0)),
        ],
        out_shape=[
            jax.ShapeDtypeStruct((n, d), jnp.float32),
            jax.ShapeDtypeStruct((1, d), jnp.float32),
        ],
        interpret=interpret,
    )(a, h_n, h_g, nb.reshape(nblocks, 1, bm), w2, w3, fcnw, fcnb2, fcgw, fcgb2)


def kernel(h_n_l, h_g_l, node_matrix, node_batch, W_2, W_3,
           fc_n_w, fc_n_b, fc_g_w, fc_g_b):
    d = h_n_l.shape[1]
    bm = 400
    fcnb2 = fc_n_b.reshape(1, d)
    fcgb2 = fc_g_b.reshape(1, d)
    for _ in range(2):
        h_n_l, h_g_l = _one_depth(h_n_l, h_g_l, node_matrix, node_batch,
                                  W_2, W_3, fc_n_w, fcnb2, fc_g_w, fcgb2,
                                  bm=bm)
    return (h_n_l, h_g_l)
